# baseline (speedup 1.0000x reference)
import functools

import jax
import jax.numpy as jnp
from jax import lax
from jax.experimental import pallas as pl
from jax.experimental.pallas import tpu as pltpu
from jax.experimental.pallas import tpu_sc as plsc

K = 64
NEG = float('-inf')
BIGI = 2**30

SEG = 1024
SEGW = SEG * 16
NSEG = 32768 // SEG
GRP = 128
NGRP = 32768 // GRP
NSTR = 8
CAPR = 1024
CAPS = CAPR // NSTR
U2 = 2


def _lexmaxmin(av, ai, bv, bi):
    gt = (av > bv) | ((av == bv) & (ai < bi))
    return (jnp.where(gt, av, bv), jnp.where(gt, ai, bi),
            jnp.where(gt, bv, av), jnp.where(gt, bi, ai))


def _sort16_desc(pairs):
    for k in (2, 4, 8, 16):
        j = k // 2
        while j >= 1:
            for i in range(16):
                p = i ^ j
                if p > i:
                    hv, hi, lv, li = _lexmaxmin(*pairs[i], *pairs[p])
                    if (i & k) == 0:
                        pairs[i], pairs[p] = (hv, hi), (lv, li)
                    else:
                        pairs[i], pairs[p] = (lv, li), (hv, hi)
            j //= 2
    return pairs


def _sc_body(x_hbm, vals_hbm, idx_hbm,
             seg0, seg1, spm, cval, cidx, rval, ridx,
             semA0, semA1, semB0, semB1):
    sid = lax.axis_index("s")
    wid = sid * 2 + lax.axis_index("c")
    ninf = jnp.full((16,), NEG, jnp.float32)
    bigi = jnp.full((16,), BIGI, jnp.int32)
    zero = jnp.zeros((16,), jnp.int32)
    lane = lax.iota(jnp.int32, 16)

    def fill_rows(ref_v, ref_i, n):
        def bodyf(i, _):
            ref_v[pl.ds(i * 16, 16)] = ninf
            ref_i[pl.ds(i * 16, 16)] = bigi
            return 0
        lax.fori_loop(0, n, bodyf, 0)

    def merge_block(blk):
        pairs = []
        for u in range(16):
            off = (blk * 16 + u) * 16
            pairs.append((cval[pl.ds(off, 16)], cidx[pl.ds(off, 16)]))
        pairs = _sort16_desc(pairs)
        for u in range(16):
            off = (48 + u) * 16
            hv, hi, _lv, _li = _lexmaxmin(
                rval[pl.ds(off, 16)], ridx[pl.ds(off, 16)], *pairs[15 - u])
            rval[pl.ds(off, 16)] = hv
            ridx[pl.ds(off, 16)] = hi
        def stage(s, _):
            j = 32 >> s
            jm1 = j - 1
            for p in range(32):
                i = ((p & ~jm1) << 1) | (p & jm1)
                ii = i * 16
                pp = ii + j * 16
                hv, hi, lv, li = _lexmaxmin(
                    rval[pl.ds(ii, 16)], ridx[pl.ds(ii, 16)],
                    rval[pl.ds(pp, 16)], ridx[pl.ds(pp, 16)])
                rval[pl.ds(ii, 16)] = hv
                ridx[pl.ds(ii, 16)] = hi
                rval[pl.ds(pp, 16)] = lv
                ridx[pl.ds(pp, 16)] = li
            return 0
        lax.fori_loop(0, 6, stage, 0)

    def run_segments(b, process, carry):
        def hop1(seg, p):
            dst = spm.at[sid, p]
            sem = semA0 if p == 0 else semA1
            pltpu.async_copy(x_hbm.at[b, pl.ds(seg * SEGW, SEGW)], dst, sem)

        def hop1_dyn(seg, p):
            dst = spm.at[sid, p]
            sem = semA0 if p == 0 else semA1
            pltpu.async_copy(x_hbm.at[b, pl.ds(seg * SEGW, SEGW)], dst, sem)

        def waitA(p):
            sem = semA0 if p == 0 else semA1
            pltpu.make_async_copy(x_hbm.at[0, pl.ds(0, SEGW)],
                                  spm.at[sid, p], sem).wait()

        def hop2(p):
            src = spm.at[sid, p]
            dst = seg0 if p == 0 else seg1
            sem = semB0 if p == 0 else semB1
            pltpu.async_copy(src, dst, sem)

        def waitB(p):
            dst = seg0 if p == 0 else seg1
            sem = semB0 if p == 0 else semB1
            pltpu.make_async_copy(x_hbm.at[0, pl.ds(0, SEGW)], dst,
                                  sem).wait()

        hop1(0, 0)
        hop1(1, 1)

        def pair_body(i, carry):
            waitA(0)
            hop2(0)
            waitB(0)

            @pl.when(i < (NSEG // 2 - 1))
            def _():
                hop1_dyn(2 * i + 2, 0)
            carry = process(seg0, 2 * i, carry)
            waitA(1)
            hop2(1)
            waitB(1)

            @pl.when(i < (NSEG // 2 - 1))
            def _():
                hop1_dyn(2 * i + 3, 1)
            carry = process(seg1, 2 * i + 1, carry)
            return carry

        return lax.fori_loop(0, NSEG // 2, pair_body, carry)

    def batch_body(bi, _):
        b = wid * 2 + bi

        def seg_max(buf):
            def grp_body(g, s):
                def row_body(j, accs):
                    base = (g * GRP + j * 16) * 16
                    accs = list(accs)
                    for u in range(16):
                        v = buf[pl.ds(base + u * 16, 16)]
                        accs[u % 4] = jnp.maximum(accs[u % 4], v)
                    return tuple(accs)
                a0, a1, a2, a3 = lax.fori_loop(
                    0, GRP // 16, row_body, (ninf, ninf, ninf, ninf))
                acc = jnp.maximum(jnp.maximum(a0, a1), jnp.maximum(a2, a3))
                s1, s2 = s
                return (jnp.maximum(s1, acc),
                        jnp.maximum(s2, jnp.minimum(s1, acc)))
            s = lax.fori_loop(0, SEG // GRP, grp_body, (ninf, ninf))
            return s[1]

        def p1_process(buf, seg, trun):
            return jnp.minimum(trun, seg_max(buf))

        t = run_segments(b, p1_process,
                         jnp.full((16,), jnp.inf, jnp.float32))

        fill_rows(rval, ridx, K)
        fill_rows(cval, cidx, CAPR)

        def seg_compact(buf, seg, state):
            addrs, nvecs = state

            def row_loop(i, carry):
                addrs = list(carry[0:NSTR])
                nvecs = list(carry[NSTR:2 * NSTR])
                for k in range(NSTR):
                    limit = (k + 1) * CAPS * 16
                    base = (k * (SEG // NSTR)) * 16
                    for u in range(U2):
                        off = base + (i * U2 + u) * 16
                        v = buf[pl.ds(off, 16)]
                        m = (v >= t) & (addrs[k] < limit)
                        plsc.store_scatter(cval, [addrs[k]], v, mask=m)
                        plsc.store_scatter(cidx, [addrs[k]], nvecs[k], mask=m)
                        addrs[k] = addrs[k] + jnp.where(m, 16, 0)
                        nvecs[k] = nvecs[k] + 1
                return tuple(addrs) + tuple(nvecs)

            out = lax.fori_loop(0, SEG // (NSTR * U2), row_loop,
                                tuple(addrs) + tuple(nvecs))
            addrs = list(out[0:NSTR])
            nvecs = [nv + (SEG - SEG // NSTR) for nv in out[NSTR:2 * NSTR]]
            return addrs, nvecs

        addrs = [lane + k * CAPS * 16 for k in range(NSTR)]
        nvecs = [zero + (k * (SEG // NSTR)) for k in range(NSTR)]

        def p2_process(buf, seg, carry):
            state = (carry[0:NSTR], carry[NSTR:2 * NSTR])
            a, nv = seg_compact(buf, seg, state)
            return tuple(a) + tuple(nv)

        out = run_segments(b, p2_process, tuple(addrs) + tuple(nvecs))
        addrs = out[0:NSTR]

        maxa = [jnp.max(a) for a in addrs]

        def csel_body(blk, _):
            k = blk >> 3
            ma = maxa[NSTR - 1]
            for kk in range(NSTR - 2, -1, -1):
                ma = jnp.where(k == kk, maxa[kk], ma)

            @pl.when(ma > blk * 256)
            def _():
                merge_block(blk)
            return 0
        lax.fori_loop(0, CAPR // 16, csel_body, 0)

        pltpu.sync_copy(rval, vals_hbm.at[b])
        pltpu.sync_copy(ridx, idx_hbm.at[b])
        return 0

    lax.fori_loop(0, 2, batch_body, 0)


def _sc_topk(xf):
    B = xf.shape[0]
    mesh = plsc.VectorSubcoreMesh(core_axis_name="c", subcore_axis_name="s")
    f = functools.partial(
        pl.kernel,
        mesh=mesh,
        compiler_params=pltpu.CompilerParams(needs_layout_passes=False),
        out_type=(
            jax.ShapeDtypeStruct((B, K * 16), jnp.float32),
            jax.ShapeDtypeStruct((B, K * 16), jnp.int32),
        ),
        scratch_types=[
            pltpu.VMEM((SEGW,), jnp.float32),
            pltpu.VMEM((SEGW,), jnp.float32),
            pltpu.VMEM_SHARED((16, 2, SEGW), jnp.float32),
            pltpu.VMEM((CAPR * 16,), jnp.float32),
            pltpu.VMEM((CAPR * 16,), jnp.int32),
            pltpu.VMEM((K * 16,), jnp.float32),
            pltpu.VMEM((K * 16,), jnp.int32),
            pltpu.SemaphoreType.DMA,
            pltpu.SemaphoreType.DMA,
            pltpu.SemaphoreType.DMA,
            pltpu.SemaphoreType.DMA,
        ],
    )(_sc_body)
    return f(xf)


def kernel(x):
    B, N, Q = x.shape
    xf = x.reshape(B, N * Q)
    v, i = _sc_topk(xf)
    return v.reshape(B, K, Q), i.reshape(B, K, Q)


# speedup vs baseline: 1.1134x; 1.1134x over previous
import functools

import jax
import jax.numpy as jnp
from jax import lax
from jax.experimental import pallas as pl
from jax.experimental.pallas import tpu as pltpu
from jax.experimental.pallas import tpu_sc as plsc

K = 64
NEG = float('-inf')
BIGI = 2**30

SEG = 2048
SEGW = SEG * 16
NSEG = 32768 // SEG
GRP = 128
NGRP = 32768 // GRP
NSTR = 8
CAPR = 512
CAPS = CAPR // NSTR
U2 = 4


def _lexmaxmin(av, ai, bv, bi):
    gt = (av > bv) | ((av == bv) & (ai < bi))
    return (jnp.where(gt, av, bv), jnp.where(gt, ai, bi),
            jnp.where(gt, bv, av), jnp.where(gt, bi, ai))


def _sort16_desc(pairs):
    for k in (2, 4, 8, 16):
        j = k // 2
        while j >= 1:
            for i in range(16):
                p = i ^ j
                if p > i:
                    hv, hi, lv, li = _lexmaxmin(*pairs[i], *pairs[p])
                    if (i & k) == 0:
                        pairs[i], pairs[p] = (hv, hi), (lv, li)
                    else:
                        pairs[i], pairs[p] = (lv, li), (hv, hi)
            j //= 2
    return pairs


def _sc_body(x_hbm, vals_hbm, idx_hbm,
             seg0, seg1, cval, cidx, rval, ridx, sem0, sem1):
    wid = lax.axis_index("s") * 2 + lax.axis_index("c")
    ninf = jnp.full((16,), NEG, jnp.float32)
    bigi = jnp.full((16,), BIGI, jnp.int32)
    zero = jnp.zeros((16,), jnp.int32)
    lane = lax.iota(jnp.int32, 16)

    def fill_rows(ref_v, ref_i, n):
        def bodyf(i, _):
            ref_v[pl.ds(i * 16, 16)] = ninf
            ref_i[pl.ds(i * 16, 16)] = bigi
            return 0
        lax.fori_loop(0, n, bodyf, 0)

    def merge_block(blk):
        pairs = []
        for u in range(16):
            off = (blk * 16 + u) * 16
            pairs.append((cval[pl.ds(off, 16)], cidx[pl.ds(off, 16)]))
        pairs = _sort16_desc(pairs)
        for u in range(16):
            off = (48 + u) * 16
            hv, hi, _lv, _li = _lexmaxmin(
                rval[pl.ds(off, 16)], ridx[pl.ds(off, 16)], *pairs[15 - u])
            rval[pl.ds(off, 16)] = hv
            ridx[pl.ds(off, 16)] = hi
        def stage(s, _):
            j = 32 >> s
            jm1 = j - 1
            for p in range(32):
                i = ((p & ~jm1) << 1) | (p & jm1)
                ii = i * 16
                pp = ii + j * 16
                hv, hi, lv, li = _lexmaxmin(
                    rval[pl.ds(ii, 16)], ridx[pl.ds(ii, 16)],
                    rval[pl.ds(pp, 16)], ridx[pl.ds(pp, 16)])
                rval[pl.ds(ii, 16)] = hv
                ridx[pl.ds(ii, 16)] = hi
                rval[pl.ds(pp, 16)] = lv
                ridx[pl.ds(pp, 16)] = li
            return 0
        lax.fori_loop(0, 6, stage, 0)

    def batch_body(bi, _):
        b = wid * 2 + bi

        def seg_max(buf):
            def grp_body(g, s):
                def row_body(j, accs):
                    base = (g * GRP + j * 32) * 16
                    accs = list(accs)
                    for u in range(32):
                        v = buf[pl.ds(base + u * 16, 16)]
                        accs[u % 4] = jnp.maximum(accs[u % 4], v)
                    return tuple(accs)
                a0, a1, a2, a3 = lax.fori_loop(
                    0, GRP // 32, row_body, (ninf, ninf, ninf, ninf))
                acc = jnp.maximum(jnp.maximum(a0, a1), jnp.maximum(a2, a3))
                out = []
                for sv in s:
                    out.append(jnp.maximum(sv, acc))
                    acc = jnp.minimum(sv, acc)
                return tuple(out)
            s = lax.fori_loop(0, SEG // GRP, grp_body,
                              (ninf, ninf, ninf, ninf))
            return s[3]

        cp = pltpu.async_copy(x_hbm.at[b, pl.ds(0, SEGW)], seg0, sem0)
        cp1 = pltpu.async_copy(x_hbm.at[b, pl.ds(SEGW, SEGW)], seg1, sem1)

        def wait0():
            pltpu.make_async_copy(x_hbm.at[0, pl.ds(0, SEGW)], seg0,
                                  sem0).wait()

        def wait1():
            pltpu.make_async_copy(x_hbm.at[0, pl.ds(0, SEGW)], seg1,
                                  sem1).wait()

        def p1_body(i, trun):
            wait0()
            t0 = seg_max(seg0)

            @pl.when(i < (NSEG // 2 - 1))
            def _():
                pltpu.async_copy(
                    x_hbm.at[b, pl.ds((2 * i + 2) * SEGW, SEGW)], seg0, sem0)
            wait1()
            t1 = seg_max(seg1)

            @pl.when(i < (NSEG // 2 - 1))
            def _():
                pltpu.async_copy(
                    x_hbm.at[b, pl.ds((2 * i + 3) * SEGW, SEGW)], seg1, sem1)
            return jnp.minimum(trun, jnp.minimum(t0, t1))

        t = lax.fori_loop(0, NSEG // 2, p1_body,
                          jnp.full((16,), jnp.inf, jnp.float32))

        fill_rows(rval, ridx, K)
        fill_rows(cval, cidx, CAPR)

        def seg_compact(buf, seg, state):
            addrs, nvecs = state

            def row_loop(i, carry):
                addrs = list(carry[0:NSTR])
                nvecs = list(carry[NSTR:2 * NSTR])
                for k in range(NSTR):
                    limit = (k + 1) * CAPS * 16
                    base = (k * (SEG // NSTR)) * 16
                    for u in range(U2):
                        off = base + (i * U2 + u) * 16
                        v = buf[pl.ds(off, 16)]
                        m = (v >= t) & (addrs[k] < limit)
                        plsc.store_scatter(cval, [addrs[k]], v, mask=m)
                        plsc.store_scatter(cidx, [addrs[k]], nvecs[k], mask=m)
                        addrs[k] = addrs[k] + jnp.where(m, 16, 0)
                        nvecs[k] = nvecs[k] + 1
                return tuple(addrs) + tuple(nvecs)

            out = lax.fori_loop(0, SEG // (NSTR * U2), row_loop,
                                tuple(addrs) + tuple(nvecs))
            addrs = list(out[0:NSTR])
            nvecs = [nv + (SEG - SEG // NSTR) for nv in out[NSTR:2 * NSTR]]
            return addrs, nvecs

        addrs = [lane + k * CAPS * 16 for k in range(NSTR)]
        nvecs = [zero + (k * (SEG // NSTR)) for k in range(NSTR)]

        cp = pltpu.async_copy(x_hbm.at[b, pl.ds(0, SEGW)], seg0, sem0)
        cp1 = pltpu.async_copy(x_hbm.at[b, pl.ds(SEGW, SEGW)], seg1, sem1)

        def p2_body(i, carry):
            state = (carry[0:NSTR], carry[NSTR:2 * NSTR])
            wait0()
            addrs, nvecs = seg_compact(seg0, 2 * i, state)

            @pl.when(i < (NSEG // 2 - 1))
            def _():
                pltpu.async_copy(
                    x_hbm.at[b, pl.ds((2 * i + 2) * SEGW, SEGW)], seg0, sem0)
            wait1()
            addrs, nvecs = seg_compact(seg1, 2 * i + 1, (addrs, nvecs))

            @pl.when(i < (NSEG // 2 - 1))
            def _():
                pltpu.async_copy(
                    x_hbm.at[b, pl.ds((2 * i + 3) * SEGW, SEGW)], seg1, sem1)
            return tuple(addrs) + tuple(nvecs)

        out = lax.fori_loop(0, NSEG // 2, p2_body, tuple(addrs) + tuple(nvecs))
        addrs = out[0:NSTR]

        maxa = [jnp.max(a) for a in addrs]

        def csel_body(blk, _):
            k = blk >> 2
            ma = maxa[NSTR - 1]
            for kk in range(NSTR - 2, -1, -1):
                ma = jnp.where(k == kk, maxa[kk], ma)

            @pl.when(ma > blk * 256)
            def _():
                merge_block(blk)
            return 0
        lax.fori_loop(0, CAPR // 16, csel_body, 0)

        pltpu.sync_copy(rval, vals_hbm.at[b])
        pltpu.sync_copy(ridx, idx_hbm.at[b])
        return 0

    lax.fori_loop(0, 2, batch_body, 0)


def _sc_topk(xf):
    B = xf.shape[0]
    mesh = plsc.VectorSubcoreMesh(core_axis_name="c", subcore_axis_name="s")
    f = functools.partial(
        pl.kernel,
        mesh=mesh,
        compiler_params=pltpu.CompilerParams(needs_layout_passes=False),
        out_type=(
            jax.ShapeDtypeStruct((B, K * 16), jnp.float32),
            jax.ShapeDtypeStruct((B, K * 16), jnp.int32),
        ),
        scratch_types=[
            pltpu.VMEM((SEGW,), jnp.float32),
            pltpu.VMEM((SEGW,), jnp.float32),
            pltpu.VMEM((CAPR * 16,), jnp.float32),
            pltpu.VMEM((CAPR * 16,), jnp.int32),
            pltpu.VMEM((K * 16,), jnp.float32),
            pltpu.VMEM((K * 16,), jnp.int32),
            pltpu.SemaphoreType.DMA,
            pltpu.SemaphoreType.DMA,
        ],
    )(_sc_body)
    return f(xf)


def kernel(x):
    B, N, Q = x.shape
    xf = x.reshape(B, N * Q)
    v, i = _sc_topk(xf)
    return v.reshape(B, K, Q), i.reshape(B, K, Q)
